# Initial kernel scaffold; baseline (speedup 1.0000x reference)
import jax
import jax.numpy as jnp
from jax.experimental import pallas as pl


def kernel(x, edge_index, W_l, b_l, W_r):
    raise NotImplementedError("write your pallas kernel here")


# speedup vs baseline: 4.3304x; 4.3304x over previous
import functools

import jax
import jax.numpy as jnp
from jax import lax
from jax.experimental import pallas as pl
from jax.experimental.pallas import tpu as pltpu
from jax.experimental.pallas import tpu_sc as plsc

N_NODES = 10000
N_EDGES = 320000
D_IN = 128
D_OUT = 128
D_AUG = 144

NC = 2
NS = 16
NW = NC * NS
EDGES_PER_TILE = N_EDGES // NW
CHUNK = 80
NCHUNK = EDGES_PER_TILE // CHUNK
N_PAD = 10240
ROWS_PER_TILE = N_PAD // NS
ZROWS = 128


def _sc_aggregate(x_aug, src, dst):
  mesh = plsc.VectorSubcoreMesh(core_axis_name="c", subcore_axis_name="s")

  @functools.partial(
      pl.kernel,
      mesh=mesh,
      compiler_params=pltpu.CompilerParams(use_tc_tiling_on_sc=False),
      out_type=jax.ShapeDtypeStruct((NC, N_PAD, D_AUG), jnp.float32),
      scratch_types=[
          pltpu.VMEM((CHUNK,), jnp.int32),
          pltpu.VMEM((CHUNK,), jnp.int32),
          pltpu.VMEM((CHUNK, D_AUG), jnp.float32),
          pltpu.VMEM((ZROWS, D_AUG), jnp.float32),
          pltpu.VMEM_SHARED((N_PAD, D_AUG), jnp.float32),
          pltpu.SemaphoreType.DMA,
      ],
  )
  def body(xaug_hbm, src_hbm, dst_hbm, out_hbm, sidx, didx, rows, zbuf, acc,
           sem):
    c = lax.axis_index("c")
    s = lax.axis_index("s")
    wid = s * NC + c

    def zero_row(r, carry):
      for k in range(D_AUG // 16):
        zbuf[r, pl.ds(k * 16, 16)] = jnp.zeros((16,), jnp.float32)
      return carry

    lax.fori_loop(0, ZROWS, zero_row, 0)

    def zero_acc(i, carry):
      pltpu.sync_copy(
          zbuf, acc.at[pl.ds(s * ROWS_PER_TILE + i * ZROWS, ZROWS)])
      return carry

    lax.fori_loop(0, ROWS_PER_TILE // ZROWS, zero_acc, 0)
    plsc.subcore_barrier()

    base = wid * EDGES_PER_TILE

    def step(i, carry):
      off = base + i * CHUNK
      pltpu.sync_copy(src_hbm.at[pl.ds(off, CHUNK)], sidx)
      pltpu.sync_copy(dst_hbm.at[pl.ds(off, CHUNK)], didx)
      pltpu.async_copy(xaug_hbm.at[sidx], rows, sem).wait()
      pltpu.sync_copy(rows, acc.at[didx], add=True)
      return carry

    lax.fori_loop(0, NCHUNK, step, 0)
    plsc.subcore_barrier()

    def out_step(i, carry):
      r0 = s * ROWS_PER_TILE + i * ZROWS
      pltpu.sync_copy(acc.at[pl.ds(r0, ZROWS)], zbuf)
      pltpu.sync_copy(zbuf, out_hbm.at[c, pl.ds(r0, ZROWS)])
      return carry

    lax.fori_loop(0, ROWS_PER_TILE // ZROWS, out_step, 0)

  return body(x_aug, src, dst)


BLK = 1000


def _combine(parts, x, w_comb, w_r_t):
  def body(p_ref, x_ref, wc_ref, wr_ref, o_ref):
    acc = p_ref[0] + p_ref[1]
    o_ref[...] = jnp.dot(
        acc, wc_ref[...], preferred_element_type=jnp.float32,
        precision=lax.Precision.HIGHEST) + jnp.dot(
            x_ref[...], wr_ref[...], preferred_element_type=jnp.float32,
            precision=lax.Precision.HIGHEST)

  return pl.pallas_call(
      body,
      grid=(N_NODES // BLK,),
      in_specs=[
          pl.BlockSpec((NC, BLK, D_AUG), lambda i: (0, i, 0)),
          pl.BlockSpec((BLK, D_IN), lambda i: (i, 0)),
          pl.BlockSpec((D_AUG, D_OUT), lambda i: (0, 0)),
          pl.BlockSpec((D_IN, D_OUT), lambda i: (0, 0)),
      ],
      out_specs=pl.BlockSpec((BLK, D_OUT), lambda i: (i, 0)),
      out_shape=jax.ShapeDtypeStruct((N_NODES, D_OUT), jnp.float32),
  )(parts, x, w_comb, w_r_t)


def kernel(x, edge_index, W_l, b_l, W_r):
  src = edge_index[0].astype(jnp.int32)
  dst = edge_index[1].astype(jnp.int32)
  x_aug = jnp.concatenate(
      [x, jnp.ones((N_NODES, 1), jnp.float32),
       jnp.zeros((N_NODES, D_AUG - D_IN - 1), jnp.float32)], axis=1)
  parts = _sc_aggregate(x_aug, src, dst)
  w_comb = jnp.concatenate(
      [W_l.T, b_l[None, :],
       jnp.zeros((D_AUG - D_IN - 1, D_OUT), jnp.float32)], axis=0)
  return _combine(parts, x, w_comb, W_r.T)


# speedup vs baseline: 7.1735x; 1.6566x over previous
import functools

import jax
import jax.numpy as jnp
from jax import lax
from jax.experimental import pallas as pl
from jax.experimental.pallas import tpu as pltpu
from jax.experimental.pallas import tpu_sc as plsc

N_NODES = 10000
N_EDGES = 320000
D_IN = 128
D_OUT = 128
D_AUG = 144

NC = 2
NS = 16
NW = NC * NS
EDGES_PER_TILE = N_EDGES // NW
CHUNK = 80
NCHUNK = EDGES_PER_TILE // CHUNK
N_PAD = 10240
ROWS_PER_TILE = N_PAD // NS
ZROWS = 128


def _sc_aggregate(x_aug, src, dst):
  mesh = plsc.VectorSubcoreMesh(core_axis_name="c", subcore_axis_name="s")

  @functools.partial(
      pl.kernel,
      mesh=mesh,
      compiler_params=pltpu.CompilerParams(use_tc_tiling_on_sc=False),
      out_type=jax.ShapeDtypeStruct((NC, N_PAD, D_AUG), jnp.float32),
      scratch_types=[
          pltpu.VMEM((2, CHUNK), jnp.int32),
          pltpu.VMEM((2, CHUNK), jnp.int32),
          pltpu.VMEM((2, CHUNK), jnp.int32),
          pltpu.VMEM((2, CHUNK), jnp.int32),
          pltpu.VMEM((CHUNK, D_AUG), jnp.float32),
          pltpu.VMEM((CHUNK, D_AUG), jnp.float32),
          pltpu.VMEM_SHARED((N_PAD, D_AUG), jnp.float32),
          pltpu.SemaphoreType.DMA,
          pltpu.SemaphoreType.DMA,
          pltpu.SemaphoreType.DMA,
          pltpu.SemaphoreType.DMA,
          pltpu.SemaphoreType.DMA,
          pltpu.SemaphoreType.DMA,
      ],
  )
  def body(xaug_hbm, eidx_hbm, out_hbm, e0, e1, e2, e3, rows0, rows1, acc,
           is0, is1, is2, is3, gs0, gs1):
    c = lax.axis_index("c")
    s = lax.axis_index("s")
    wid = s * NC + c

    ebufs = (e0, e1, e2, e3)
    isems = (is0, is1, is2, is3)
    rbufs = (rows0, rows1)
    gsems = (gs0, gs1)

    def zero_row(r, carry):
      for k in range(D_AUG // 16):
        rows0[r, pl.ds(k * 16, 16)] = jnp.zeros((16,), jnp.float32)
      return carry

    lax.fori_loop(0, CHUNK, zero_row, 0)

    def zero_acc(i, carry):
      pltpu.sync_copy(
          rows0, acc.at[pl.ds(s * ROWS_PER_TILE + i * CHUNK, CHUNK)])
      return carry

    lax.fori_loop(0, ROWS_PER_TILE // CHUNK, zero_acc, 0)
    plsc.subcore_barrier()

    def start_idx(j, ib):
      pltpu.async_copy(eidx_hbm.at[wid, j], ebufs[ib], isems[ib])

    def wait_idx(ib):
      pltpu.make_async_copy(eidx_hbm.at[0, 0], ebufs[ib], isems[ib]).wait()

    def start_gather(ib, rb):
      pltpu.async_copy(xaug_hbm.at[ebufs[ib].at[0]], rbufs[rb], gsems[rb])

    def wait_gather(rb):
      pltpu.make_async_copy(xaug_hbm.at[pl.ds(0, CHUNK)], rbufs[rb],
                            gsems[rb]).wait()

    pltpu.sync_copy(eidx_hbm.at[wid, 0], e0)
    start_gather(0, 0)
    start_idx(1, 1)

    def quad(p, carry):
      j0 = 4 * p
      for b in range(4):
        j = j0 + b
        ib = b
        rb = b % 2

        @pl.when(j < NCHUNK)
        def _():
          @pl.when(j + 1 < NCHUNK)
          def _():
            wait_idx((ib + 1) % 4)
            start_gather((ib + 1) % 4, 1 - rb)

          @pl.when(j + 2 < NCHUNK)
          def _():
            start_idx(j + 2, (ib + 2) % 4)

          wait_gather(rb)
          pltpu.sync_copy(rbufs[rb], acc.at[ebufs[ib].at[1]], add=True)

      return carry

    lax.fori_loop(0, (NCHUNK + 3) // 4, quad, 0)
    plsc.subcore_barrier()

    def out_step(i, carry):
      r0 = s * ROWS_PER_TILE + i * CHUNK
      pltpu.sync_copy(acc.at[pl.ds(r0, CHUNK)], rows0)
      pltpu.sync_copy(rows0, out_hbm.at[c, pl.ds(r0, CHUNK)])
      return carry

    lax.fori_loop(0, ROWS_PER_TILE // CHUNK, out_step, 0)

  eidx = jnp.stack(
      [src.reshape(NW, NCHUNK, CHUNK), dst.reshape(NW, NCHUNK, CHUNK)],
      axis=2)
  return body(x_aug, eidx)


BLK = 1000


def _combine(parts, x, w_comb, w_r_t):
  def body(p_ref, x_ref, wc_ref, wr_ref, o_ref):
    acc = p_ref[0] + p_ref[1]
    o_ref[...] = jnp.dot(
        acc, wc_ref[...], preferred_element_type=jnp.float32,
        precision=lax.Precision.HIGHEST) + jnp.dot(
            x_ref[...], wr_ref[...], preferred_element_type=jnp.float32,
            precision=lax.Precision.HIGHEST)

  return pl.pallas_call(
      body,
      grid=(N_NODES // BLK,),
      in_specs=[
          pl.BlockSpec((NC, BLK, D_AUG), lambda i: (0, i, 0)),
          pl.BlockSpec((BLK, D_IN), lambda i: (i, 0)),
          pl.BlockSpec((D_AUG, D_OUT), lambda i: (0, 0)),
          pl.BlockSpec((D_IN, D_OUT), lambda i: (0, 0)),
      ],
      out_specs=pl.BlockSpec((BLK, D_OUT), lambda i: (i, 0)),
      out_shape=jax.ShapeDtypeStruct((N_NODES, D_OUT), jnp.float32),
  )(parts, x, w_comb, w_r_t)


def kernel(x, edge_index, W_l, b_l, W_r):
  src = edge_index[0].astype(jnp.int32)
  dst = edge_index[1].astype(jnp.int32)
  x_aug = jnp.concatenate(
      [x, jnp.ones((N_NODES, 1), jnp.float32),
       jnp.zeros((N_NODES, D_AUG - D_IN - 1), jnp.float32)], axis=1)
  parts = _sc_aggregate(x_aug, src, dst)
  w_comb = jnp.concatenate(
      [W_l.T, b_l[None, :],
       jnp.zeros((D_AUG - D_IN - 1, D_OUT), jnp.float32)], axis=0)
  return _combine(parts, x, w_comb, W_r.T)
